# Initial kernel scaffold; baseline (speedup 1.0000x reference)
import jax
import jax.numpy as jnp
from jax.experimental import pallas as pl


def kernel(x, edge_index, w1, b1, w2, b2, gamma, beta):
    raise NotImplementedError("write your pallas kernel here")


# speedup vs baseline: 4.6002x; 4.6002x over previous
import functools

import jax
import jax.numpy as jnp
from jax import lax
from jax.experimental import pallas as pl
from jax.experimental.pallas import tpu as pltpu
from jax.experimental.pallas import tpu_sc as plsc

N = 10000
E = 320000
D = 128
H = 256
HOP = 4
EPS = 1e-5

NC = 2
NS = 16
COLS = D // NC
NP = 10240
RPT = NP // NS
RSUB = 128
NSUB = RPT // RSUB
EPT = E // NS
CH = 128
NFULL = EPT // CH
REM = EPT - NFULL * CH


def _prop_body(x_hbm, src_hbm, dst_hbm, hops_hbm,
               gbuf, acc,
               schunk, dchunk, rows, stage, zeros, normf):
  c = lax.axis_index("c")
  s = lax.axis_index("s")
  row0 = s * RPT
  e0 = s * EPT

  z16f = jnp.zeros((16,), jnp.float32)
  o16f = jnp.ones((16,), jnp.float32)

  def _fill_zeros(r, _):
    for q in range(COLS // 16):
      zeros[r, pl.ds(q * 16, 16)] = z16f
    return 0
  lax.fori_loop(0, RSUB // 4, _fill_zeros, 0)

  def _zero_acc(r0):
    for h in range(4):
      pltpu.sync_copy(zeros, acc.at[pl.ds(r0 + h * (RSUB // 4), RSUB // 4)])

  def _fill_ones(r, _):
    for q in range(COLS // 16):
      rows[r, pl.ds(q * 16, 16)] = o16f
    return 0
  lax.fori_loop(0, CH, _fill_ones, 0)

  def _pad_schunk():
    for i in range((CH - REM) // 16):
      schunk[pl.ds(REM + i * 16, 16)] = jnp.zeros((16,), jnp.int32)

  def _pad_dchunk():
    for i in range((CH - REM) // 16):
      dchunk[pl.ds(REM + i * 16, 16)] = jnp.full((16,), NP, jnp.int32)

  for sub in range(NSUB):
    _zero_acc(row0 + sub * RSUB)

  plsc.subcore_barrier()

  def _deg_chunk(j, _):
    off = e0 + pl.multiple_of(j * CH, CH)
    pltpu.sync_copy(dst_hbm.at[pl.ds(off, CH)], dchunk)
    pltpu.sync_copy(rows, acc.at[dchunk], add=True)
    return 0
  lax.fori_loop(0, NFULL, _deg_chunk, 0)
  pltpu.sync_copy(dst_hbm.at[pl.ds(e0 + NFULL * CH, REM)],
                  dchunk.at[pl.ds(0, REM)])
  _pad_dchunk()
  pltpu.sync_copy(rows, acc.at[dchunk], add=True)

  plsc.subcore_barrier()

  for sub in range(NSUB):
    r0 = row0 + sub * RSUB
    pltpu.sync_copy(acc.at[pl.ds(r0, RSUB)], stage)

    def _norm_row(r, _, sub=sub):
      dv = stage[r, pl.ds(0, 16)]
      dm = jnp.maximum(dv, 1.0)
      ii = lax.bitcast_convert_type(dm, jnp.int32)
      ii = jnp.int32(0x5F3759DF) - lax.shift_right_arithmetic(ii, 1)
      y = lax.bitcast_convert_type(ii, jnp.float32)
      for _ in range(4):
        y = y * (1.5 - 0.5 * dm * y * y)
      noff = pl.multiple_of((sub * RSUB + r) * 16, 16)
      normf[pl.ds(noff, 16)] = jnp.where(dv > 0.5, y, 0.0)
      return 0
    lax.fori_loop(0, RSUB, _norm_row, 0)
    _zero_acc(r0)

  for sub in range(NSUB):
    r0 = row0 + sub * RSUB
    pltpu.sync_copy(x_hbm.at[c, pl.ds(r0, RSUB)], stage)

    def _scale_x(r, _, sub=sub):
      nv = normf[pl.ds(pl.multiple_of((sub * RSUB + r) * 16, 16), 16)]
      for q in range(COLS // 16):
        stage[r, pl.ds(q * 16, 16)] = stage[r, pl.ds(q * 16, 16)] * nv
      return 0
    lax.fori_loop(0, RSUB, _scale_x, 0)
    pltpu.sync_copy(stage, gbuf.at[pl.ds(r0, RSUB)])

  plsc.subcore_barrier()

  for k in range(HOP):
    def _edge_chunk(j, _):
      off = e0 + pl.multiple_of(j * CH, CH)
      pltpu.sync_copy(src_hbm.at[pl.ds(off, CH)], schunk)
      pltpu.sync_copy(dst_hbm.at[pl.ds(off, CH)], dchunk)
      pltpu.sync_copy(gbuf.at[schunk], rows)
      pltpu.sync_copy(rows, acc.at[dchunk], add=True)
      return 0
    lax.fori_loop(0, NFULL, _edge_chunk, 0)
    pltpu.sync_copy(src_hbm.at[pl.ds(e0 + NFULL * CH, REM)],
                    schunk.at[pl.ds(0, REM)])
    pltpu.sync_copy(dst_hbm.at[pl.ds(e0 + NFULL * CH, REM)],
                    dchunk.at[pl.ds(0, REM)])
    _pad_schunk()
    _pad_dchunk()
    pltpu.sync_copy(gbuf.at[schunk], rows)
    pltpu.sync_copy(rows, acc.at[dchunk], add=True)

    plsc.subcore_barrier()

    for sub in range(NSUB):
      r0 = row0 + sub * RSUB
      pltpu.sync_copy(acc.at[pl.ds(r0, RSUB)], stage)

      def _scale_o(r, _, sub=sub):
        nv = normf[pl.ds(pl.multiple_of((sub * RSUB + r) * 16, 16), 16)]
        for q in range(COLS // 16):
          stage[r, pl.ds(q * 16, 16)] = stage[r, pl.ds(q * 16, 16)] * nv
        return 0
      lax.fori_loop(0, RSUB, _scale_o, 0)
      pltpu.sync_copy(stage, hops_hbm.at[k, c, pl.ds(r0, RSUB)])

      if k < HOP - 1:
        lax.fori_loop(0, RSUB, _scale_o, 0)
        pltpu.sync_copy(stage, gbuf.at[pl.ds(r0, RSUB)])
        _zero_acc(r0)

    if k < HOP - 1:
      plsc.subcore_barrier()


_prop_kernel = functools.partial(
    pl.kernel,
    out_type=jax.ShapeDtypeStruct((HOP, NC, NP, COLS), jnp.float32),
    mesh=plsc.VectorSubcoreMesh(core_axis_name="c", subcore_axis_name="s",
                                num_cores=NC, num_subcores=NS),
    scratch_types=[
        pltpu.VMEM_SHARED((NP, COLS), jnp.float32),
        pltpu.VMEM_SHARED((NP + 8, COLS), jnp.float32),
        pltpu.VMEM((CH,), jnp.int32),
        pltpu.VMEM((CH,), jnp.int32),
        pltpu.VMEM((CH, COLS), jnp.float32),
        pltpu.VMEM((RSUB, COLS), jnp.float32),
        pltpu.VMEM((RSUB // 4, COLS), jnp.float32),
        pltpu.VMEM((RPT * 16,), jnp.float32),
    ],
)(_prop_body)


BR = 1024


def _ffn_body(hops_ref, w1_ref, b1_ref, w2_ref, b2_ref, gm_ref, bt_ref,
              out_ref, r_ref):
  hblk = hops_ref[...]
  hcat = jnp.concatenate([hblk[:, 0], hblk[:, 1]], axis=-1)
  m = jnp.max(hcat, axis=0)
  y1 = jnp.maximum(
      jnp.dot(m, w1_ref[...], preferred_element_type=jnp.float32)
      + b1_ref[...], 0.0)
  y = jnp.dot(y1, w2_ref[...], preferred_element_type=jnp.float32) + b2_ref[...]
  z = m + y
  mu = jnp.mean(z, axis=1, keepdims=True)
  zc = z - mu
  var = jnp.mean(zc * zc, axis=1, keepdims=True)
  out_ref[...] = zc * lax.rsqrt(var + EPS) * gm_ref[...] + bt_ref[...]
  r_ref[...] = m


_ffn_call = pl.pallas_call(
    _ffn_body,
    grid=(NP // BR,),
    in_specs=[
        pl.BlockSpec((HOP, NC, BR, COLS), lambda i: (0, 0, i, 0)),
        pl.BlockSpec((D, H), lambda i: (0, 0)),
        pl.BlockSpec((1, H), lambda i: (0, 0)),
        pl.BlockSpec((H, D), lambda i: (0, 0)),
        pl.BlockSpec((1, D), lambda i: (0, 0)),
        pl.BlockSpec((1, D), lambda i: (0, 0)),
        pl.BlockSpec((1, D), lambda i: (0, 0)),
    ],
    out_specs=[
        pl.BlockSpec((BR, D), lambda i: (i, 0)),
        pl.BlockSpec((BR, D), lambda i: (i, 0)),
    ],
    out_shape=[
        jax.ShapeDtypeStruct((NP, D), jnp.float32),
        jax.ShapeDtypeStruct((NP, D), jnp.float32),
    ],
)


@jax.jit
def kernel(x, edge_index, w1, b1, w2, b2, gamma, beta):
  src = edge_index[0]
  dst = edge_index[1]
  xs = jnp.pad(x, ((0, NP - N), (0, 0)))
  xsplit = jnp.stack([xs[:, :COLS], xs[:, COLS:]])
  hops = _prop_kernel(xsplit, src, dst)
  rst_ff, r = _ffn_call(hops, w1, b1.reshape(1, H), w2, b2.reshape(1, D),
                        gamma.reshape(1, D), beta.reshape(1, D))
  return (rst_ff[:N], r[:N])


# speedup vs baseline: 5.8719x; 1.2764x over previous
import functools

import jax
import jax.numpy as jnp
from jax import lax
from jax.experimental import pallas as pl
from jax.experimental.pallas import tpu as pltpu
from jax.experimental.pallas import tpu_sc as plsc

N = 10000
E = 320000
D = 128
H = 256
HOP = 4
EPS = 1e-5

NC = 2
NS = 16
COLS = D // NC
NP = 10240
RPT = NP // NS
RSUB = 128
NSUB = RPT // RSUB
CH = 128
NCH = E // CH
CPT = NCH // NS
G = 4


def _prop_body(x_hbm, src_hbm, dst_hbm, hops_hbm,
               gbuf, acc,
               sgrp, dgrp, rows, stage, zeros, normf):
  c = lax.axis_index("c")
  s = lax.axis_index("s")
  row0 = s * RPT
  cbase = s * CPT
  ngrp = jnp.where(s == NS - 1, (NCH - (NS - 1) * CPT) // G, CPT // G)

  z16f = jnp.zeros((16,), jnp.float32)
  o16f = jnp.ones((16,), jnp.float32)

  def _fill_zeros(r, _):
    for q in range(COLS // 16):
      zeros[r, pl.ds(q * 16, 16)] = z16f
    return 0
  lax.fori_loop(0, RSUB // 8, _fill_zeros, 0)

  def _zero_acc(r0):
    for h in range(8):
      pltpu.sync_copy(zeros, acc.at[pl.ds(r0 + h * (RSUB // 8), RSUB // 8)])

  def _fill_ones(r, _):
    for q in range(COLS // 16):
      rows[r, pl.ds(q * 16, 16)] = o16f
    return 0
  lax.fori_loop(0, CH, _fill_ones, 0)

  for sub in range(NSUB):
    _zero_acc(row0 + sub * RSUB)

  plsc.subcore_barrier()

  def _deg_group(g, _):
    goff = cbase + g * G
    pltpu.sync_copy(dst_hbm.at[pl.ds(goff, G)], dgrp)
    for b in range(G):
      pltpu.sync_copy(rows, acc.at[dgrp.at[b]], add=True)
    return 0
  lax.fori_loop(0, ngrp, _deg_group, 0)

  plsc.subcore_barrier()

  for sub in range(NSUB):
    r0 = row0 + sub * RSUB
    pltpu.sync_copy(acc.at[pl.ds(r0, RSUB)], stage)

    def _norm_row(r, _, sub=sub):
      dv = stage[r, pl.ds(0, 16)]
      dm = jnp.maximum(dv, 1.0)
      ii = lax.bitcast_convert_type(dm, jnp.int32)
      ii = jnp.int32(0x5F3759DF) - lax.shift_right_arithmetic(ii, 1)
      y = lax.bitcast_convert_type(ii, jnp.float32)
      for _ in range(4):
        y = y * (1.5 - 0.5 * dm * y * y)
      noff = pl.multiple_of((sub * RSUB + r) * 16, 16)
      normf[pl.ds(noff, 16)] = jnp.where(dv > 0.5, y, 0.0)
      return 0
    lax.fori_loop(0, RSUB, _norm_row, 0)
    _zero_acc(r0)

  for sub in range(NSUB):
    r0 = row0 + sub * RSUB
    pltpu.sync_copy(x_hbm.at[c, pl.ds(r0, RSUB)], stage)

    def _scale_x(r, _, sub=sub):
      nv = normf[pl.ds(pl.multiple_of((sub * RSUB + r) * 16, 16), 16)]
      for q in range(COLS // 16):
        stage[r, pl.ds(q * 16, 16)] = stage[r, pl.ds(q * 16, 16)] * nv
      return 0
    lax.fori_loop(0, RSUB, _scale_x, 0)
    pltpu.sync_copy(stage, gbuf.at[pl.ds(r0, RSUB)])

  plsc.subcore_barrier()

  for k in range(HOP):
    def _edge_group(g, _):
      goff = cbase + g * G
      pltpu.sync_copy(src_hbm.at[pl.ds(goff, G)], sgrp)
      pltpu.sync_copy(dst_hbm.at[pl.ds(goff, G)], dgrp)
      for b in range(G):
        pltpu.sync_copy(gbuf.at[sgrp.at[b]], rows)
        pltpu.sync_copy(rows, acc.at[dgrp.at[b]], add=True)
      return 0
    lax.fori_loop(0, ngrp, _edge_group, 0)

    plsc.subcore_barrier()

    for sub in range(NSUB):
      r0 = row0 + sub * RSUB
      pltpu.sync_copy(acc.at[pl.ds(r0, RSUB)], stage)

      def _scale_o(r, _, sub=sub):
        nv = normf[pl.ds(pl.multiple_of((sub * RSUB + r) * 16, 16), 16)]
        for q in range(COLS // 16):
          stage[r, pl.ds(q * 16, 16)] = stage[r, pl.ds(q * 16, 16)] * nv
        return 0
      lax.fori_loop(0, RSUB, _scale_o, 0)
      pltpu.sync_copy(stage, hops_hbm.at[k, c, pl.ds(r0, RSUB)])

      if k < HOP - 1:
        lax.fori_loop(0, RSUB, _scale_o, 0)
        pltpu.sync_copy(stage, gbuf.at[pl.ds(r0, RSUB)])
        _zero_acc(r0)

    if k < HOP - 1:
      plsc.subcore_barrier()


_prop_kernel = functools.partial(
    pl.kernel,
    out_type=jax.ShapeDtypeStruct((HOP, NC, NP, COLS), jnp.float32),
    mesh=plsc.VectorSubcoreMesh(core_axis_name="c", subcore_axis_name="s",
                                num_cores=NC, num_subcores=NS),
    scratch_types=[
        pltpu.VMEM_SHARED((NP, COLS), jnp.float32),
        pltpu.VMEM_SHARED((NP + 8, COLS), jnp.float32),
        pltpu.VMEM((G, CH), jnp.int32),
        pltpu.VMEM((G, CH), jnp.int32),
        pltpu.VMEM((CH, COLS), jnp.float32),
        pltpu.VMEM((RSUB, COLS), jnp.float32),
        pltpu.VMEM((RSUB // 8, COLS), jnp.float32),
        pltpu.VMEM((RPT * 16,), jnp.float32),
    ],
)(_prop_body)


BR = 1024


def _ffn_body(hops_ref, w1_ref, b1_ref, w2_ref, b2_ref, gm_ref, bt_ref,
              out_ref, r_ref):
  hblk = hops_ref[...]
  hcat = jnp.concatenate([hblk[:, 0], hblk[:, 1]], axis=-1)
  m = jnp.max(hcat, axis=0)
  y1 = jnp.maximum(
      jnp.dot(m, w1_ref[...], preferred_element_type=jnp.float32)
      + b1_ref[...], 0.0)
  y = jnp.dot(y1, w2_ref[...], preferred_element_type=jnp.float32) + b2_ref[...]
  z = m + y
  mu = jnp.mean(z, axis=1, keepdims=True)
  zc = z - mu
  var = jnp.mean(zc * zc, axis=1, keepdims=True)
  out_ref[...] = zc * lax.rsqrt(var + EPS) * gm_ref[...] + bt_ref[...]
  r_ref[...] = m


_ffn_call = pl.pallas_call(
    _ffn_body,
    grid=(NP // BR,),
    in_specs=[
        pl.BlockSpec((HOP, NC, BR, COLS), lambda i: (0, 0, i, 0)),
        pl.BlockSpec((D, H), lambda i: (0, 0)),
        pl.BlockSpec((1, H), lambda i: (0, 0)),
        pl.BlockSpec((H, D), lambda i: (0, 0)),
        pl.BlockSpec((1, D), lambda i: (0, 0)),
        pl.BlockSpec((1, D), lambda i: (0, 0)),
        pl.BlockSpec((1, D), lambda i: (0, 0)),
    ],
    out_specs=[
        pl.BlockSpec((BR, D), lambda i: (i, 0)),
        pl.BlockSpec((BR, D), lambda i: (i, 0)),
    ],
    out_shape=[
        jax.ShapeDtypeStruct((NP, D), jnp.float32),
        jax.ShapeDtypeStruct((NP, D), jnp.float32),
    ],
)


@jax.jit
def kernel(x, edge_index, w1, b1, w2, b2, gamma, beta):
  src = edge_index[0].reshape(NCH, CH)
  dst = edge_index[1].reshape(NCH, CH)
  xs = jnp.pad(x, ((0, NP - N), (0, 0)))
  xsplit = jnp.stack([xs[:, :COLS], xs[:, COLS:]])
  hops = _prop_kernel(xsplit, src, dst)
  rst_ff, r = _ffn_call(hops, w1, b1.reshape(1, H), w2, b2.reshape(1, D),
                        gamma.reshape(1, D), beta.reshape(1, D))
  return (rst_ff[:N], r[:N])


# speedup vs baseline: 7.2577x; 1.2360x over previous
import functools

import jax
import jax.numpy as jnp
from jax import lax
from jax.experimental import pallas as pl
from jax.experimental.pallas import tpu as pltpu
from jax.experimental.pallas import tpu_sc as plsc

N = 10000
E = 320000
D = 128
H = 256
HOP = 4
EPS = 1e-5

NC = 2
NS = 16
COLS = D // NC
NP = 10240
RPT = NP // NS
RSUB = 80
NSUB = RPT // RSUB
CH = 64
NCH = E // CH
CPT = NCH // NS
G = 8


def _prop_body(x_hbm, src_hbm, dst_hbm, hops_hbm,
               gbuf, acc,
               sgrp, dgrp, rows, stage, zeros, normf,
               semg0, semg1, sems0, sems1):
  c = lax.axis_index("c")
  s = lax.axis_index("s")
  row0 = s * RPT
  cbase = s * CPT
  ngrp = jnp.where(s == NS - 1, (NCH - (NS - 1) * CPT) // G, CPT // G)

  z16f = jnp.zeros((16,), jnp.float32)
  o16f = jnp.ones((16,), jnp.float32)

  def _fill_zeros(r, _):
    for q in range(COLS // 16):
      zeros[r, pl.ds(q * 16, 16)] = z16f
    return 0
  lax.fori_loop(0, 16, _fill_zeros, 0)

  def _zero_acc(r0):
    for h in range(RSUB // 16):
      pltpu.sync_copy(zeros, acc.at[pl.ds(r0 + h * 16, 16)])

  def _fill_ones(r, _):
    for q in range(COLS // 16):
      rows[0, r, pl.ds(q * 16, 16)] = o16f
    return 0
  lax.fori_loop(0, CH, _fill_ones, 0)

  for sub in range(NSUB):
    _zero_acc(row0 + sub * RSUB)

  plsc.subcore_barrier()

  def _deg_group(g, _):
    goff = cbase + g * G
    pltpu.sync_copy(dst_hbm.at[pl.ds(goff, G)], dgrp)
    for b in range(G):
      pltpu.sync_copy(rows.at[0], acc.at[dgrp.at[b]], add=True)
    return 0
  lax.fori_loop(0, ngrp, _deg_group, 0)

  plsc.subcore_barrier()

  for sub in range(NSUB):
    r0 = row0 + sub * RSUB
    pltpu.sync_copy(acc.at[pl.ds(r0, RSUB)], stage)

    def _norm_row(r, _, sub=sub):
      dv = stage[r, pl.ds(0, 16)]
      dm = jnp.maximum(dv, 1.0)
      ii = lax.bitcast_convert_type(dm, jnp.int32)
      ii = jnp.int32(0x5F3759DF) - lax.shift_right_arithmetic(ii, 1)
      y = lax.bitcast_convert_type(ii, jnp.float32)
      for _ in range(4):
        y = y * (1.5 - 0.5 * dm * y * y)
      noff = pl.multiple_of((sub * RSUB + r) * 16, 16)
      normf[pl.ds(noff, 16)] = jnp.where(dv > 0.5, y, 0.0)
      return 0
    lax.fori_loop(0, RSUB, _norm_row, 0)
    _zero_acc(r0)

  for sub in range(NSUB):
    r0 = row0 + sub * RSUB
    pltpu.sync_copy(x_hbm.at[c, pl.ds(r0, RSUB)], stage)

    def _scale_x(r, _, sub=sub):
      nv = normf[pl.ds(pl.multiple_of((sub * RSUB + r) * 16, 16), 16)]
      for q in range(COLS // 16):
        stage[r, pl.ds(q * 16, 16)] = stage[r, pl.ds(q * 16, 16)] * nv
      return 0
    lax.fori_loop(0, RSUB, _scale_x, 0)
    pltpu.sync_copy(stage, gbuf.at[pl.ds(r0, RSUB)])

  plsc.subcore_barrier()

  semg = (semg0, semg1)
  sems = (sems0, sems1)

  for k in range(HOP):
    def _edge_group(g, _):
      goff = cbase + g * G
      pltpu.sync_copy(src_hbm.at[pl.ds(goff, G)], sgrp)
      pltpu.sync_copy(dst_hbm.at[pl.ds(goff, G)], dgrp)
      gd = [None, None]
      sd = [None, None]
      gd[0] = pltpu.async_copy(gbuf.at[sgrp.at[0]], rows.at[0], semg[0])
      for b in range(G):
        p = b % 2
        q = (b + 1) % 2
        gd[p].wait()
        if b + 1 < G:
          if sd[q] is not None:
            sd[q].wait()
          gd[q] = pltpu.async_copy(gbuf.at[sgrp.at[b + 1]], rows.at[q],
                                   semg[q])
        sd[p] = pltpu.async_copy(rows.at[p], acc.at[dgrp.at[b]], sems[p],
                                 add=True)
      sd[0].wait()
      sd[1].wait()
      return 0
    lax.fori_loop(0, ngrp, _edge_group, 0)

    plsc.subcore_barrier()

    for sub in range(NSUB):
      r0 = row0 + sub * RSUB
      pltpu.sync_copy(acc.at[pl.ds(r0, RSUB)], stage)

      def _scale_o(r, _, sub=sub):
        nv = normf[pl.ds(pl.multiple_of((sub * RSUB + r) * 16, 16), 16)]
        for q in range(COLS // 16):
          stage[r, pl.ds(q * 16, 16)] = stage[r, pl.ds(q * 16, 16)] * nv
        return 0
      lax.fori_loop(0, RSUB, _scale_o, 0)
      pltpu.sync_copy(stage, hops_hbm.at[k, c, pl.ds(r0, RSUB)])

      if k < HOP - 1:
        lax.fori_loop(0, RSUB, _scale_o, 0)
        pltpu.sync_copy(stage, gbuf.at[pl.ds(r0, RSUB)])
        _zero_acc(r0)

    if k < HOP - 1:
      plsc.subcore_barrier()


_prop_kernel = functools.partial(
    pl.kernel,
    out_type=jax.ShapeDtypeStruct((HOP, NC, NP, COLS), jnp.float32),
    mesh=plsc.VectorSubcoreMesh(core_axis_name="c", subcore_axis_name="s",
                                num_cores=NC, num_subcores=NS),
    scratch_types=[
        pltpu.VMEM_SHARED((NP, COLS), jnp.float32),
        pltpu.VMEM_SHARED((NP + 8, COLS), jnp.float32),
        pltpu.VMEM((G, CH), jnp.int32),
        pltpu.VMEM((G, CH), jnp.int32),
        pltpu.VMEM((2, CH, COLS), jnp.float32),
        pltpu.VMEM((RSUB, COLS), jnp.float32),
        pltpu.VMEM((16, COLS), jnp.float32),
        pltpu.VMEM((RPT * 16,), jnp.float32),
        pltpu.SemaphoreType.DMA,
        pltpu.SemaphoreType.DMA,
        pltpu.SemaphoreType.DMA,
        pltpu.SemaphoreType.DMA,
    ],
)(_prop_body)


BR = 1024


def _ffn_body(hops_ref, w1_ref, b1_ref, w2_ref, b2_ref, gm_ref, bt_ref,
              out_ref, r_ref):
  hblk = hops_ref[...]
  hcat = jnp.concatenate([hblk[:, 0], hblk[:, 1]], axis=-1)
  m = jnp.max(hcat, axis=0)
  y1 = jnp.maximum(
      jnp.dot(m, w1_ref[...], preferred_element_type=jnp.float32)
      + b1_ref[...], 0.0)
  y = jnp.dot(y1, w2_ref[...], preferred_element_type=jnp.float32) + b2_ref[...]
  z = m + y
  mu = jnp.mean(z, axis=1, keepdims=True)
  zc = z - mu
  var = jnp.mean(zc * zc, axis=1, keepdims=True)
  out_ref[...] = zc * lax.rsqrt(var + EPS) * gm_ref[...] + bt_ref[...]
  r_ref[...] = m


_ffn_call = pl.pallas_call(
    _ffn_body,
    grid=(NP // BR,),
    in_specs=[
        pl.BlockSpec((HOP, NC, BR, COLS), lambda i: (0, 0, i, 0)),
        pl.BlockSpec((D, H), lambda i: (0, 0)),
        pl.BlockSpec((1, H), lambda i: (0, 0)),
        pl.BlockSpec((H, D), lambda i: (0, 0)),
        pl.BlockSpec((1, D), lambda i: (0, 0)),
        pl.BlockSpec((1, D), lambda i: (0, 0)),
        pl.BlockSpec((1, D), lambda i: (0, 0)),
    ],
    out_specs=[
        pl.BlockSpec((BR, D), lambda i: (i, 0)),
        pl.BlockSpec((BR, D), lambda i: (i, 0)),
    ],
    out_shape=[
        jax.ShapeDtypeStruct((NP, D), jnp.float32),
        jax.ShapeDtypeStruct((NP, D), jnp.float32),
    ],
)


@jax.jit
def kernel(x, edge_index, w1, b1, w2, b2, gamma, beta):
  src = edge_index[0].reshape(NCH, CH)
  dst = edge_index[1].reshape(NCH, CH)
  xs = jnp.pad(x, ((0, NP - N), (0, 0)))
  xsplit = jnp.stack([xs[:, :COLS], xs[:, COLS:]])
  hops = _prop_kernel(xsplit, src, dst)
  rst_ff, r = _ffn_call(hops, w1, b1.reshape(1, H), w2, b2.reshape(1, D),
                        gamma.reshape(1, D), beta.reshape(1, D))
  return (rst_ff[:N], r[:N])


# speedup vs baseline: 8.2081x; 1.1310x over previous
import functools

import jax
import jax.numpy as jnp
from jax import lax
from jax.experimental import pallas as pl
from jax.experimental.pallas import tpu as pltpu
from jax.experimental.pallas import tpu_sc as plsc

N = 10000
E = 320000
D = 128
H = 256
HOP = 4
EPS = 1e-5

NC = 2
NS = 16
COLS = D // NC
NP = 10240
RPT = NP // NS
RSUB = 80
NSUB = RPT // RSUB
CH = 64
NCH = E // CH
CPT = NCH // NS
G = 8


def _prop_body(x_hbm, eb_hbm, hops_hbm,
               gbuf, acc,
               ebuf, rows, stage, zeros, normf,
               semg0, semg1, sems0, sems1, semi0, semi1):
  c = lax.axis_index("c")
  s = lax.axis_index("s")
  row0 = s * RPT
  cbase = s * CPT
  ngrp = jnp.where(s == NS - 1, (NCH - (NS - 1) * CPT) // G, CPT // G)

  z16f = jnp.zeros((16,), jnp.float32)
  o16f = jnp.ones((16,), jnp.float32)

  semg = (semg0, semg1)
  sems = (sems0, sems1)
  semi = (semi0, semi1)

  def _fill_zeros(r, _):
    for q in range(COLS // 16):
      zeros[r, pl.ds(q * 16, 16)] = z16f
    return 0
  lax.fori_loop(0, 16, _fill_zeros, 0)

  def _zero_acc(r0):
    for h in range(RSUB // 16):
      pltpu.sync_copy(zeros, acc.at[pl.ds(r0 + h * 16, 16)])

  def _fill_ones(r, _):
    for q in range(COLS // 16):
      rows[0, r, pl.ds(q * 16, 16)] = o16f
    return 0
  lax.fori_loop(0, CH, _fill_ones, 0)

  for sub in range(NSUB):
    _zero_acc(row0 + sub * RSUB)

  plsc.subcore_barrier()

  def _deg_group(g, _):
    goff = cbase + g * G
    pltpu.sync_copy(eb_hbm.at[pl.ds(goff, G)], ebuf.at[0])
    ds = []
    for b in range(G):
      ds.append(pltpu.async_copy(rows.at[0], acc.at[ebuf.at[0, b, 1]],
                                 sems[b % 2], add=True))
    for d in ds:
      d.wait()
    return 0
  lax.fori_loop(0, ngrp, _deg_group, 0)

  plsc.subcore_barrier()

  for sub in range(NSUB):
    r0 = row0 + sub * RSUB
    pltpu.sync_copy(acc.at[pl.ds(r0, RSUB)], stage)

    def _norm_row(r, _, sub=sub):
      dv = stage[r, pl.ds(0, 16)]
      dm = jnp.maximum(dv, 1.0)
      ii = lax.bitcast_convert_type(dm, jnp.int32)
      ii = jnp.int32(0x5F3759DF) - lax.shift_right_arithmetic(ii, 1)
      y = lax.bitcast_convert_type(ii, jnp.float32)
      for _ in range(4):
        y = y * (1.5 - 0.5 * dm * y * y)
      noff = pl.multiple_of((sub * RSUB + r) * 16, 16)
      normf[pl.ds(noff, 16)] = jnp.where(dv > 0.5, y, 0.0)
      return 0
    lax.fori_loop(0, RSUB, _norm_row, 0)
    _zero_acc(r0)

  for sub in range(NSUB):
    r0 = row0 + sub * RSUB
    pltpu.sync_copy(x_hbm.at[c, pl.ds(r0, RSUB)], stage)

    def _scale_x(r, _, sub=sub):
      nv = normf[pl.ds(pl.multiple_of((sub * RSUB + r) * 16, 16), 16)]
      for q in range(COLS // 16):
        stage[r, pl.ds(q * 16, 16)] = stage[r, pl.ds(q * 16, 16)] * nv
      return 0
    lax.fori_loop(0, RSUB, _scale_x, 0)
    pltpu.sync_copy(stage, gbuf.at[pl.ds(r0, RSUB)])

  plsc.subcore_barrier()

  def _issue_idx(gg, p):
    goff = cbase + gg * G
    return pltpu.async_copy(eb_hbm.at[pl.ds(goff, G)], ebuf.at[p], semi[p])

  def _wait_idx(gg, p):
    goff = cbase + gg * G
    pltpu.make_async_copy(eb_hbm.at[pl.ds(goff, G)], ebuf.at[p],
                          semi[p]).wait()

  def _process_group(gg, p):
    _wait_idx(gg, p)
    gd = [None, None]
    sd = [None, None]
    gd[0] = pltpu.async_copy(gbuf.at[ebuf.at[p, 0, 0]], rows.at[0], semg[0])
    for b in range(G):
      pb = b % 2
      qb = (b + 1) % 2
      gd[pb].wait()
      if b + 1 < G:
        if sd[qb] is not None:
          sd[qb].wait()
        gd[qb] = pltpu.async_copy(gbuf.at[ebuf.at[p, b + 1, 0]], rows.at[qb],
                                  semg[qb])
      sd[pb] = pltpu.async_copy(rows.at[pb], acc.at[ebuf.at[p, b, 1]],
                                sems[pb], add=True)
    sd[(G - 1) % 2].wait()
    sd[G % 2].wait()

  for k in range(HOP):
    _issue_idx(0, 0)
    _issue_idx(1, 1)

    def _pair(i, _):
      for p in range(2):
        gg = 2 * i + p
        _process_group(gg, p)

        @pl.when(gg + 2 < ngrp)
        def _():
          _issue_idx(gg + 2, p)
      return 0
    lax.fori_loop(0, ngrp // 2, _pair, 0)

    @pl.when(ngrp % 2 == 1)
    def _():
      _process_group(ngrp - 1, 0)

    plsc.subcore_barrier()

    for sub in range(NSUB):
      r0 = row0 + sub * RSUB
      pltpu.sync_copy(acc.at[pl.ds(r0, RSUB)], stage)

      def _scale_o(r, _, sub=sub):
        nv = normf[pl.ds(pl.multiple_of((sub * RSUB + r) * 16, 16), 16)]
        for q in range(COLS // 16):
          stage[r, pl.ds(q * 16, 16)] = stage[r, pl.ds(q * 16, 16)] * nv
        return 0
      lax.fori_loop(0, RSUB, _scale_o, 0)
      pltpu.sync_copy(stage, hops_hbm.at[k, c, pl.ds(r0, RSUB)])

      if k < HOP - 1:
        lax.fori_loop(0, RSUB, _scale_o, 0)
        pltpu.sync_copy(stage, gbuf.at[pl.ds(r0, RSUB)])
        _zero_acc(r0)

    if k < HOP - 1:
      plsc.subcore_barrier()


_prop_kernel = functools.partial(
    pl.kernel,
    out_type=jax.ShapeDtypeStruct((HOP, NC, NP, COLS), jnp.float32),
    mesh=plsc.VectorSubcoreMesh(core_axis_name="c", subcore_axis_name="s",
                                num_cores=NC, num_subcores=NS),
    scratch_types=[
        pltpu.VMEM_SHARED((NP, COLS), jnp.float32),
        pltpu.VMEM_SHARED((NP + 8, COLS), jnp.float32),
        pltpu.VMEM((2, G, 2, CH), jnp.int32),
        pltpu.VMEM((2, CH, COLS), jnp.float32),
        pltpu.VMEM((RSUB, COLS), jnp.float32),
        pltpu.VMEM((16, COLS), jnp.float32),
        pltpu.VMEM((RPT * 16,), jnp.float32),
        pltpu.SemaphoreType.DMA,
        pltpu.SemaphoreType.DMA,
        pltpu.SemaphoreType.DMA,
        pltpu.SemaphoreType.DMA,
        pltpu.SemaphoreType.DMA,
        pltpu.SemaphoreType.DMA,
    ],
)(_prop_body)


BR = 1024


def _ffn_body(hops_ref, w1_ref, b1_ref, w2_ref, b2_ref, gm_ref, bt_ref,
              out_ref, r_ref):
  hblk = hops_ref[...]
  hcat = jnp.concatenate([hblk[:, 0], hblk[:, 1]], axis=-1)
  m = jnp.max(hcat, axis=0)
  y1 = jnp.maximum(
      jnp.dot(m, w1_ref[...], preferred_element_type=jnp.float32)
      + b1_ref[...], 0.0)
  y = jnp.dot(y1, w2_ref[...], preferred_element_type=jnp.float32) + b2_ref[...]
  z = m + y
  mu = jnp.mean(z, axis=1, keepdims=True)
  zc = z - mu
  var = jnp.mean(zc * zc, axis=1, keepdims=True)
  out_ref[...] = zc * lax.rsqrt(var + EPS) * gm_ref[...] + bt_ref[...]
  r_ref[...] = m


_ffn_call = pl.pallas_call(
    _ffn_body,
    grid=(NP // BR,),
    in_specs=[
        pl.BlockSpec((HOP, NC, BR, COLS), lambda i: (0, 0, i, 0)),
        pl.BlockSpec((D, H), lambda i: (0, 0)),
        pl.BlockSpec((1, H), lambda i: (0, 0)),
        pl.BlockSpec((H, D), lambda i: (0, 0)),
        pl.BlockSpec((1, D), lambda i: (0, 0)),
        pl.BlockSpec((1, D), lambda i: (0, 0)),
        pl.BlockSpec((1, D), lambda i: (0, 0)),
    ],
    out_specs=[
        pl.BlockSpec((BR, D), lambda i: (i, 0)),
        pl.BlockSpec((BR, D), lambda i: (i, 0)),
    ],
    out_shape=[
        jax.ShapeDtypeStruct((NP, D), jnp.float32),
        jax.ShapeDtypeStruct((NP, D), jnp.float32),
    ],
)


@jax.jit
def kernel(x, edge_index, w1, b1, w2, b2, gamma, beta):
  eb = jnp.stack([edge_index[0].reshape(NCH, CH),
                  edge_index[1].reshape(NCH, CH)], axis=1)
  xs = jnp.pad(x, ((0, NP - N), (0, 0)))
  xsplit = jnp.stack([xs[:, :COLS], xs[:, COLS:]])
  hops = _prop_kernel(xsplit, eb)
  rst_ff, r = _ffn_call(hops, w1, b1.reshape(1, H), w2, b2.reshape(1, D),
                        gamma.reshape(1, D), beta.reshape(1, D))
  return (rst_ff[:N], r[:N])


# speedup vs baseline: 8.4587x; 1.0305x over previous
import functools

import jax
import jax.numpy as jnp
from jax import lax
from jax.experimental import pallas as pl
from jax.experimental.pallas import tpu as pltpu
from jax.experimental.pallas import tpu_sc as plsc

N = 10000
E = 320000
D = 128
H = 256
HOP = 4
EPS = 1e-5

NC = 2
NS = 16
COLS = D // NC
NP = 10240
RPT = NP // NS
RSUB = 80
NSUB = RPT // RSUB
CH = 64
NCH = E // CH
CPT = NCH // NS
G = 8


def _prop_body(x_hbm, eb_hbm, hops_hbm,
               gbuf, acc,
               ebuf, rows, stage, zeros, normf,
               semg0, semg1, sems0, sems1, semi0, semi1):
  c = lax.axis_index("c")
  s = lax.axis_index("s")
  row0 = s * RPT
  cbase = s * CPT
  ngrp = jnp.where(s == NS - 1, (NCH - (NS - 1) * CPT) // G, CPT // G)

  z16f = jnp.zeros((16,), jnp.float32)
  o16f = jnp.ones((16,), jnp.float32)

  semg = (semg0, semg1)
  sems = (sems0, sems1)
  semi = (semi0, semi1)

  def _fill_zeros(r, _):
    for q in range(COLS // 16):
      zeros[r, pl.ds(q * 16, 16)] = z16f
    return 0
  lax.fori_loop(0, 16, _fill_zeros, 0)

  def _zero_acc(r0):
    for h in range(RSUB // 16):
      pltpu.sync_copy(zeros, acc.at[pl.ds(r0 + h * 16, 16)])

  def _fill_ones(r, _):
    for q in range(COLS // 16):
      rows[0, r, pl.ds(q * 16, 16)] = o16f
    return 0
  lax.fori_loop(0, CH, _fill_ones, 0)

  for sub in range(NSUB):
    _zero_acc(row0 + sub * RSUB)

  plsc.subcore_barrier()

  def _issue_idx0(gg, p):
    goff = cbase + gg * G
    return pltpu.async_copy(eb_hbm.at[pl.ds(goff, G)], ebuf.at[p], semi[p])

  def _wait_idx0(gg, p):
    goff = cbase + gg * G
    pltpu.make_async_copy(eb_hbm.at[pl.ds(goff, G)], ebuf.at[p],
                          semi[p]).wait()

  def _process_deg(gg, p):
    _wait_idx0(gg, p)
    ds = []
    for b in range(G):
      ds.append(pltpu.async_copy(rows.at[0], acc.at[ebuf.at[p, b, 1]],
                                 sems[b % 2], add=True))
    for d in ds:
      d.wait()

  _issue_idx0(0, 0)
  _issue_idx0(1, 1)

  def _deg_pair(i, _):
    for p in range(2):
      gg = 2 * i + p
      _process_deg(gg, p)

      @pl.when(gg + 2 < ngrp)
      def _():
        _issue_idx0(gg + 2, p)
    return 0
  lax.fori_loop(0, ngrp // 2, _deg_pair, 0)

  @pl.when(ngrp % 2 == 1)
  def _():
    _process_deg(ngrp - 1, 0)

  plsc.subcore_barrier()

  for sub in range(NSUB):
    r0 = row0 + sub * RSUB
    pltpu.sync_copy(acc.at[pl.ds(r0, RSUB)], stage)

    def _norm_row(r, _, sub=sub):
      dv = stage[r, pl.ds(0, 16)]
      dm = jnp.maximum(dv, 1.0)
      ii = lax.bitcast_convert_type(dm, jnp.int32)
      ii = jnp.int32(0x5F3759DF) - lax.shift_right_arithmetic(ii, 1)
      y = lax.bitcast_convert_type(ii, jnp.float32)
      for _ in range(4):
        y = y * (1.5 - 0.5 * dm * y * y)
      noff = pl.multiple_of((sub * RSUB + r) * 16, 16)
      normf[pl.ds(noff, 16)] = jnp.where(dv > 0.5, y, 0.0)
      return 0
    lax.fori_loop(0, RSUB, _norm_row, 0)
    _zero_acc(r0)

  for sub in range(NSUB):
    r0 = row0 + sub * RSUB
    pltpu.sync_copy(x_hbm.at[c, pl.ds(r0, RSUB)], stage)

    def _scale_x(r, _, sub=sub):
      nv = normf[pl.ds(pl.multiple_of((sub * RSUB + r) * 16, 16), 16)]
      for q in range(COLS // 16):
        stage[r, pl.ds(q * 16, 16)] = stage[r, pl.ds(q * 16, 16)] * nv
      return 0
    lax.fori_loop(0, RSUB, _scale_x, 0)
    pltpu.sync_copy(stage, gbuf.at[pl.ds(r0, RSUB)])

  plsc.subcore_barrier()

  def _issue_idx(gg, p):
    goff = cbase + gg * G
    return pltpu.async_copy(eb_hbm.at[pl.ds(goff, G)], ebuf.at[p], semi[p])

  def _wait_idx(gg, p):
    goff = cbase + gg * G
    pltpu.make_async_copy(eb_hbm.at[pl.ds(goff, G)], ebuf.at[p],
                          semi[p]).wait()

  def _process_group(gg, p):
    _wait_idx(gg, p)
    gd = [None, None]
    sd = [None, None]
    gd[0] = pltpu.async_copy(gbuf.at[ebuf.at[p, 0, 0]], rows.at[0], semg[0])
    for b in range(G):
      pb = b % 2
      qb = (b + 1) % 2
      gd[pb].wait()
      if b + 1 < G:
        if sd[qb] is not None:
          sd[qb].wait()
        gd[qb] = pltpu.async_copy(gbuf.at[ebuf.at[p, b + 1, 0]], rows.at[qb],
                                  semg[qb])
      sd[pb] = pltpu.async_copy(rows.at[pb], acc.at[ebuf.at[p, b, 1]],
                                sems[pb], add=True)
    sd[(G - 1) % 2].wait()
    sd[G % 2].wait()

  for k in range(HOP):
    _issue_idx(0, 0)
    _issue_idx(1, 1)

    def _pair(i, _):
      for p in range(2):
        gg = 2 * i + p
        _process_group(gg, p)

        @pl.when(gg + 2 < ngrp)
        def _():
          _issue_idx(gg + 2, p)
      return 0
    lax.fori_loop(0, ngrp // 2, _pair, 0)

    @pl.when(ngrp % 2 == 1)
    def _():
      _process_group(ngrp - 1, 0)

    plsc.subcore_barrier()

    for sub in range(NSUB):
      r0 = row0 + sub * RSUB
      pltpu.sync_copy(acc.at[pl.ds(r0, RSUB)], stage)

      def _scale_o(r, _, sub=sub):
        nv = normf[pl.ds(pl.multiple_of((sub * RSUB + r) * 16, 16), 16)]
        for q in range(COLS // 16):
          stage[r, pl.ds(q * 16, 16)] = stage[r, pl.ds(q * 16, 16)] * nv
        return 0
      lax.fori_loop(0, RSUB, _scale_o, 0)
      pltpu.sync_copy(stage, hops_hbm.at[k, c, pl.ds(r0, RSUB)])

      if k < HOP - 1:
        lax.fori_loop(0, RSUB, _scale_o, 0)
        pltpu.sync_copy(stage, gbuf.at[pl.ds(r0, RSUB)])
        _zero_acc(r0)

    if k < HOP - 1:
      plsc.subcore_barrier()


_prop_kernel = functools.partial(
    pl.kernel,
    out_type=jax.ShapeDtypeStruct((HOP, NC, NP, COLS), jnp.float32),
    mesh=plsc.VectorSubcoreMesh(core_axis_name="c", subcore_axis_name="s",
                                num_cores=NC, num_subcores=NS),
    scratch_types=[
        pltpu.VMEM_SHARED((NP, COLS), jnp.float32),
        pltpu.VMEM_SHARED((NP + 8, COLS), jnp.float32),
        pltpu.VMEM((2, G, 2, CH), jnp.int32),
        pltpu.VMEM((2, CH, COLS), jnp.float32),
        pltpu.VMEM((RSUB, COLS), jnp.float32),
        pltpu.VMEM((16, COLS), jnp.float32),
        pltpu.VMEM((RPT * 16,), jnp.float32),
        pltpu.SemaphoreType.DMA,
        pltpu.SemaphoreType.DMA,
        pltpu.SemaphoreType.DMA,
        pltpu.SemaphoreType.DMA,
        pltpu.SemaphoreType.DMA,
        pltpu.SemaphoreType.DMA,
    ],
)(_prop_body)


BR = 1024


def _ffn_body(hops_ref, w1_ref, b1_ref, w2_ref, b2_ref, gm_ref, bt_ref,
              out_ref, r_ref):
  hblk = hops_ref[...]
  hcat = jnp.concatenate([hblk[:, 0], hblk[:, 1]], axis=-1)
  m = jnp.max(hcat, axis=0)
  y1 = jnp.maximum(
      jnp.dot(m, w1_ref[...], preferred_element_type=jnp.float32)
      + b1_ref[...], 0.0)
  y = jnp.dot(y1, w2_ref[...], preferred_element_type=jnp.float32) + b2_ref[...]
  z = m + y
  mu = jnp.mean(z, axis=1, keepdims=True)
  zc = z - mu
  var = jnp.mean(zc * zc, axis=1, keepdims=True)
  out_ref[...] = zc * lax.rsqrt(var + EPS) * gm_ref[...] + bt_ref[...]
  r_ref[...] = m


_ffn_call = pl.pallas_call(
    _ffn_body,
    grid=(NP // BR,),
    in_specs=[
        pl.BlockSpec((HOP, NC, BR, COLS), lambda i: (0, 0, i, 0)),
        pl.BlockSpec((D, H), lambda i: (0, 0)),
        pl.BlockSpec((1, H), lambda i: (0, 0)),
        pl.BlockSpec((H, D), lambda i: (0, 0)),
        pl.BlockSpec((1, D), lambda i: (0, 0)),
        pl.BlockSpec((1, D), lambda i: (0, 0)),
        pl.BlockSpec((1, D), lambda i: (0, 0)),
    ],
    out_specs=[
        pl.BlockSpec((BR, D), lambda i: (i, 0)),
        pl.BlockSpec((BR, D), lambda i: (i, 0)),
    ],
    out_shape=[
        jax.ShapeDtypeStruct((NP, D), jnp.float32),
        jax.ShapeDtypeStruct((NP, D), jnp.float32),
    ],
)


@jax.jit
def kernel(x, edge_index, w1, b1, w2, b2, gamma, beta):
  eb = jnp.stack([edge_index[0].reshape(NCH, CH),
                  edge_index[1].reshape(NCH, CH)], axis=1)
  xs = jnp.pad(x, ((0, NP - N), (0, 0)))
  xsplit = jnp.stack([xs[:, :COLS], xs[:, COLS:]])
  hops = _prop_kernel(xsplit, eb)
  rst_ff, r = _ffn_call(hops, w1, b1.reshape(1, H), w2, b2.reshape(1, D),
                        gamma.reshape(1, D), beta.reshape(1, D))
  return (rst_ff[:N], r[:N])
